# baseline (speedup 1.0000x reference)
import functools

import jax
import jax.numpy as jnp
from jax import lax
from jax.experimental import pallas as pl
from jax.experimental.pallas import tpu as pltpu
from jax.experimental.pallas import tpu_sc as plsc

N = 100000
P = 1000
A = 8

NC = 2
NS = 16
NW = NC * NS
N2 = 100352
NQ = 4
QSZ = N2 // NQ
RQ = QSZ // NW
CH = 56
NCH = RQ // CH

B1 = 2000
G1 = N // B1
BC = 3136
GC = QSZ // BC

f32 = jnp.float32


def _blockdiag(w, reps):
    return jnp.kron(jnp.eye(reps, dtype=w.dtype), w)


def _full(shape):
    return pl.BlockSpec(shape, lambda i: tuple(0 for _ in shape))


def _sc_gather_body(q):
    def body(tab_hbm, pid_hbm, out_hbm, vbuf, ibuf, sem):
        c = lax.axis_index("c")
        s = lax.axis_index("s")
        wid = s * NC + c
        row0 = pl.multiple_of(q * QSZ + wid * RQ, 8)
        loc0 = pl.multiple_of(wid * RQ, 8)

        def step(k, carry):
            off = pl.multiple_of(row0 + k * CH, 8)
            loc = pl.multiple_of(loc0 + k * CH, 8)
            pltpu.sync_copy(pid_hbm.at[pl.ds(off, CH)], ibuf)
            pltpu.async_copy(tab_hbm.at[ibuf], vbuf, sem).wait()
            pltpu.sync_copy(vbuf, out_hbm.at[pl.ds(loc, CH)])
            return carry

        lax.fori_loop(0, NCH, step, 0)

    return body


def _sc_gather_chunk(tab, pid_pad, q):
    k = pl.kernel(
        _sc_gather_body(q),
        out_type=jax.ShapeDtypeStruct((QSZ, 512), f32),
        mesh=plsc.VectorSubcoreMesh(core_axis_name="c", subcore_axis_name="s"),
        scratch_types=[
            pltpu.VMEM((CH, 512), f32),
            pltpu.VMEM((CH,), jnp.int32),
            pltpu.SemaphoreType.DMA,
        ],
        name=f"sc_gather_q{q}",
    )
    return k(tab, pid_pad)


def _passA(x_ref, pid_ref, wb1_ref, bb1_ref, wb2e_ref, bb2e_ref,
           meas_ref, sums1_ref):
    x = x_ref[...]
    x = jnp.where(jnp.isnan(x), 0.0, x)
    x = jnp.where(x == jnp.inf, 1.0, x)
    x = jnp.where(x == -jnp.inf, -1.0, x)
    h = jnp.maximum(
        lax.dot_general(x, wb1_ref[...], (((1,), (0,)), ((), ())),
                        preferred_element_type=f32) + bb1_ref[...], 0.0)
    meas = lax.dot_general(h, wb2e_ref[...], (((1,), (0,)), ((), ())),
                           preferred_element_type=f32) + bb2e_ref[...]
    meas_ref[...] = meas

    pid = pid_ref[0]
    iot = lax.broadcasted_iota(jnp.int32, (P, pid.shape[1]), 0)
    mask = (iot == pid).astype(f32)
    part = lax.dot_general(mask, meas, (((1,), (0,)), ((), ())),
                           preferred_element_type=f32,
                           precision=lax.Precision.HIGHEST)

    @pl.when(pl.program_id(0) == 0)
    def _init():
        sums1_ref[...] = jnp.zeros_like(sums1_ref)

    sums1_ref[...] += part


def _passB(meas_ref, pid_ref, t1_ref, wg1_ref, bg1_ref,
           h1_ref, cnt_ref, sumsu_ref):
    pid = pid_ref[0]
    Bn = pid.shape[1]
    iot = lax.broadcasted_iota(jnp.int32, (P, Bn), 0)
    mask = (iot == pid).astype(f32)

    g1row = lax.dot_general(mask, t1_ref[...], (((0,), (0,)), ((), ())),
                            preferred_element_type=f32,
                            precision=lax.Precision.HIGHEST)
    cnt = jnp.maximum(g1row[:, 24:25], 1.0)
    pm = meas_ref[...][:, :24] + g1row[:, :24] / cnt
    h1 = jnp.maximum(
        lax.dot_general(pm, wg1_ref[...], (((1,), (0,)), ((), ())),
                        preferred_element_type=f32) + bg1_ref[...], 0.0)
    h1_ref[...] = h1
    cnt_ref[...] = cnt

    part = lax.dot_general(mask, h1, (((1,), (0,)), ((), ())),
                           preferred_element_type=f32,
                           precision=lax.Precision.HIGHEST)

    @pl.when(pl.program_id(0) == 0)
    def _init():
        sumsu_ref[...] = jnp.zeros_like(sumsu_ref)

    sumsu_ref[...] += part


def _passC(h1_ref, g2_ref, cnt_ref, wg2_ref, bg2_ref, wgo_ref, bgo_ref,
           wc1_ref, bc1_ref,
           ln1g_ref, ln1b_ref, wc2_ref, bc2_ref, ln2g_ref, ln2b_ref,
           wc3_ref, bc3_ref, out_ref):
    cnt = jnp.maximum(cnt_ref[...], 1.0)
    p2 = h1_ref[...] + g2_ref[...] / cnt
    wg2 = wg2_ref[...]
    parts = [
        lax.dot_general(p2[:, 64 * a:64 * (a + 1)], wg2,
                        (((1,), (0,)), ((), ())), preferred_element_type=f32)
        for a in range(A)
    ]
    h2 = jnp.maximum(jnp.concatenate(parts, axis=1) + bg2_ref[...], 0.0)
    agg = lax.dot_general(h2, wgo_ref[...], (((1,), (0,)), ((), ())),
                          preferred_element_type=f32) + bgo_ref[...]
    gm = (agg[:, 0:3] + agg[:, 3:6] + agg[:, 6:9] + agg[:, 9:12]
          + agg[:, 12:15] + agg[:, 15:18] + agg[:, 18:21]
          + agg[:, 21:24]) * (1.0 / A)
    c = lax.dot_general(gm, wc1_ref[...], (((1,), (0,)), ((), ())),
                        preferred_element_type=f32) + bc1_ref[...]
    mu = jnp.mean(c, axis=-1, keepdims=True)
    var = jnp.mean((c - mu) ** 2, axis=-1, keepdims=True)
    c = (c - mu) * lax.rsqrt(var + 1e-5) * ln1g_ref[...] + ln1b_ref[...]
    c = jnp.maximum(c, 0.0)
    c = lax.dot_general(c, wc2_ref[...], (((1,), (0,)), ((), ())),
                        preferred_element_type=f32) + bc2_ref[...]
    mu = jnp.mean(c, axis=-1, keepdims=True)
    var = jnp.mean((c - mu) ** 2, axis=-1, keepdims=True)
    c = (c - mu) * lax.rsqrt(var + 1e-5) * ln2g_ref[...] + ln2b_ref[...]
    c = jnp.maximum(c, 0.0)
    out_ref[...] = lax.dot_general(c, wc3_ref[...], (((1,), (0,)), ((), ())),
                                   preferred_element_type=f32) + bc3_ref[...]


def _passC_chunk(q, h1, g2q, cnt_n, consts):
    (Wg2, bg2, Wgob, bgo, Wc1, bc1,
     ln1g, ln1b, Wc2, bc2, ln2g, ln2b, Wc3, bc3) = consts
    qb = q * (QSZ // BC)
    return pl.pallas_call(
        _passC,
        grid=(GC,),
        in_specs=[
            pl.BlockSpec((BC, 512), lambda i: (qb + i, 0)),
            pl.BlockSpec((BC, 512), lambda i: (i, 0)),
            pl.BlockSpec((BC, 1), lambda i: (qb + i, 0)),
            _full(Wg2.shape), _full(bg2.shape),
            _full(Wgob.shape), _full(bgo.shape),
            _full(Wc1.shape), _full(bc1.shape),
            _full((1, 64)), _full((1, 64)),
            _full(Wc2.shape), _full((1, 32)),
            _full((1, 32)), _full((1, 32)),
            _full(Wc3.shape), _full((1, 1)),
        ],
        out_specs=pl.BlockSpec((BC, 1), lambda i: (i, 0)),
        out_shape=jax.ShapeDtypeStruct((QSZ, 1), f32),
    )(h1, g2q, cnt_n, Wg2, bg2, Wgob, bgo, Wc1, bc1,
      ln1g, ln1b, Wc2, bc2, ln2g, ln2b, Wc3, bc3)


def kernel(x, partition_ids, W_emb1, b_emb1, W_emb2, b_emb2, W_g1, b_g1,
           W_g2, b_g2, W_go, b_go, W_c1, b_c1, ln1_g, ln1_b, W_c2, b_c2,
           ln2_g, ln2_b, W_c3, b_c3):
    x2 = x.reshape(N, A * x.shape[2]).astype(f32)
    pid32 = partition_ids.astype(jnp.int32)
    pid3 = pid32.reshape(G1, 1, B1)
    pid_pad = jnp.pad(pid32, (0, N2 - N))

    Wb1 = _blockdiag(W_emb1, A)
    bb1 = jnp.tile(b_emb1, A).reshape(1, -1)
    Wb2 = _blockdiag(W_emb2, A)
    Wb2e = jnp.concatenate([Wb2, jnp.zeros((Wb2.shape[0], 8), f32)], axis=1)
    bb2e = jnp.concatenate(
        [jnp.tile(b_emb2, A), jnp.ones((1,), f32), jnp.zeros((7,), f32)]
    ).reshape(1, 32)
    Wg1 = _blockdiag(W_g1, A)
    bg1 = jnp.tile(b_g1, A).reshape(1, -1)
    bg2 = jnp.tile(b_g2, A).reshape(1, -1)
    Wgob = _blockdiag(W_go, A)
    bgo = jnp.tile(b_go, A).reshape(1, -1)

    meas, sums1 = pl.pallas_call(
        _passA,
        grid=(G1,),
        in_specs=[
            pl.BlockSpec((B1, 128), lambda i: (i, 0)),
            pl.BlockSpec((1, 1, B1), lambda i: (i, 0, 0)),
            _full(Wb1.shape), _full(bb1.shape),
            _full(Wb2e.shape), _full(bb2e.shape),
        ],
        out_specs=[
            pl.BlockSpec((B1, 32), lambda i: (i, 0)),
            pl.BlockSpec((P, 32), lambda i: (0, 0)),
        ],
        out_shape=[
            jax.ShapeDtypeStruct((N, 32), f32),
            jax.ShapeDtypeStruct((P, 32), f32),
        ],
    )(x2, pid3, Wb1, bb1, Wb2e, bb2e)

    h1, cnt_n, sumsu = pl.pallas_call(
        _passB,
        grid=(G1,),
        in_specs=[
            pl.BlockSpec((B1, 32), lambda i: (i, 0)),
            pl.BlockSpec((1, 1, B1), lambda i: (i, 0, 0)),
            _full((P, 32)),
            _full(Wg1.shape), _full(bg1.shape),
        ],
        out_specs=[
            pl.BlockSpec((B1, 512), lambda i: (i, 0)),
            pl.BlockSpec((B1, 1), lambda i: (i, 0)),
            pl.BlockSpec((P, 512), lambda i: (0, 0)),
        ],
        out_shape=[
            jax.ShapeDtypeStruct((N2, 512), f32),
            jax.ShapeDtypeStruct((N2, 1), f32),
            jax.ShapeDtypeStruct((P, 512), f32),
        ],
    )(meas, pid3, sums1, Wg1, bg1)

    consts = (W_g2.astype(f32), bg2, Wgob, bgo,
              W_c1.astype(f32), b_c1.reshape(1, -1),
              ln1_g.reshape(1, -1), ln1_b.reshape(1, -1),
              W_c2.astype(f32), b_c2.reshape(1, -1),
              ln2_g.reshape(1, -1), ln2_b.reshape(1, -1),
              W_c3.astype(f32), b_c3.reshape(1, 1))

    outs = []
    for q in range(NQ):
        g2q = _sc_gather_chunk(sumsu, pid_pad, q)
        outs.append(_passC_chunk(q, h1, g2q, cnt_n, consts))

    out = jnp.concatenate(outs, axis=0)
    return out[:N, 0]


# speedup vs baseline: 1.2330x; 1.2330x over previous
import functools

import jax
import jax.numpy as jnp
from jax import lax
from jax.experimental import pallas as pl
from jax.experimental.pallas import tpu as pltpu
from jax.experimental.pallas import tpu_sc as plsc

N = 100000
P = 1000
A = 8

NC = 2
NS = 16
NW = NC * NS
N2 = 100352
NQ = 4
QSZ = N2 // NQ
RQ = QSZ // NW
CH = 56
NCH = RQ // CH

B1 = 2000
G1 = N // B1
BC = 3136
GC = QSZ // BC

f32 = jnp.float32


def _blockdiag(w, reps):
    return jnp.kron(jnp.eye(reps, dtype=w.dtype), w)


def _full(shape):
    return pl.BlockSpec(shape, lambda i: tuple(0 for _ in shape))


def _sc_gather_body(q):
    def body(tab_hbm, pid_hbm, out_hbm, vbuf, ibuf, sem):
        c = lax.axis_index("c")
        s = lax.axis_index("s")
        wid = s * NC + c
        row0 = pl.multiple_of(q * QSZ + wid * RQ, 8)
        loc0 = pl.multiple_of(wid * RQ, 8)

        def step(k, carry):
            off = pl.multiple_of(row0 + k * CH, 8)
            loc = pl.multiple_of(loc0 + k * CH, 8)
            pltpu.sync_copy(pid_hbm.at[pl.ds(off, CH)], ibuf)
            pltpu.async_copy(tab_hbm.at[ibuf], vbuf, sem).wait()
            pltpu.sync_copy(vbuf, out_hbm.at[pl.ds(loc, CH)])
            return carry

        lax.fori_loop(0, NCH, step, 0)

    return body


def _sc_gather_chunk(tab, pid_pad, q):
    k = pl.kernel(
        _sc_gather_body(q),
        out_type=jax.ShapeDtypeStruct((QSZ, 512), f32),
        mesh=plsc.VectorSubcoreMesh(core_axis_name="c", subcore_axis_name="s"),
        scratch_types=[
            pltpu.VMEM((CH, 512), f32),
            pltpu.VMEM((CH,), jnp.int32),
            pltpu.SemaphoreType.DMA,
        ],
        name=f"sc_gather_q{q}",
    )
    return k(tab, pid_pad)


def _exact_mask_dot(mask_bf, vals, dims):
    hi = vals.astype(jnp.bfloat16)
    r1 = vals - hi.astype(f32)
    mid = r1.astype(jnp.bfloat16)
    lo = (r1 - mid.astype(f32)).astype(jnp.bfloat16)
    out = lax.dot_general(mask_bf, hi, dims, preferred_element_type=f32)
    out += lax.dot_general(mask_bf, mid, dims, preferred_element_type=f32)
    out += lax.dot_general(mask_bf, lo, dims, preferred_element_type=f32)
    return out


def _passA(x_ref, pid_ref, wb1_ref, bb1_ref, wb2e_ref, bb2e_ref,
           meas_ref, sums1_ref):
    x = x_ref[...]
    x = jnp.where(jnp.isnan(x), 0.0, x)
    x = jnp.where(x == jnp.inf, 1.0, x)
    x = jnp.where(x == -jnp.inf, -1.0, x)
    h = jnp.maximum(
        lax.dot_general(x, wb1_ref[...], (((1,), (0,)), ((), ())),
                        preferred_element_type=f32) + bb1_ref[...], 0.0)
    meas = lax.dot_general(h, wb2e_ref[...], (((1,), (0,)), ((), ())),
                           preferred_element_type=f32) + bb2e_ref[...]
    meas_ref[...] = meas

    pid = pid_ref[0]
    iot = lax.broadcasted_iota(jnp.int32, (P, pid.shape[1]), 0)
    mask = (iot == pid).astype(jnp.bfloat16)
    part = _exact_mask_dot(mask, meas, (((1,), (0,)), ((), ())))

    @pl.when(pl.program_id(0) == 0)
    def _init():
        sums1_ref[...] = jnp.zeros_like(sums1_ref)

    sums1_ref[...] += part


def _passB(meas_ref, pid_ref, t1_ref, wg1_ref, bg1_ref,
           h1_ref, cnt_ref, sumsu_ref):
    pid = pid_ref[0]
    Bn = pid.shape[1]
    iot = lax.broadcasted_iota(jnp.int32, (P, Bn), 0)
    mask = (iot == pid).astype(jnp.bfloat16)

    g1row = _exact_mask_dot(mask, t1_ref[...],
                            (((0,), (0,)), ((), ())))
    cnt = jnp.maximum(g1row[:, 24:25], 1.0)
    pm = meas_ref[...][:, :24] + g1row[:, :24] / cnt
    h1 = jnp.maximum(
        lax.dot_general(pm, wg1_ref[...], (((1,), (0,)), ((), ())),
                        preferred_element_type=f32) + bg1_ref[...], 0.0)
    h1_ref[...] = h1
    cnt_ref[...] = cnt

    part = _exact_mask_dot(mask, h1, (((1,), (0,)), ((), ())))

    @pl.when(pl.program_id(0) == 0)
    def _init():
        sumsu_ref[...] = jnp.zeros_like(sumsu_ref)

    sumsu_ref[...] += part


def _passC(h1_ref, g2_ref, cnt_ref, wg2_ref, bg2_ref, wgo_ref, bgo_ref,
           wc1_ref, bc1_ref,
           ln1g_ref, ln1b_ref, wc2_ref, bc2_ref, ln2g_ref, ln2b_ref,
           wc3_ref, bc3_ref, out_ref):
    cnt = jnp.maximum(cnt_ref[...], 1.0)
    p2 = h1_ref[...] + g2_ref[...] / cnt
    wg2 = wg2_ref[...]
    parts = [
        lax.dot_general(p2[:, 64 * a:64 * (a + 1)], wg2,
                        (((1,), (0,)), ((), ())), preferred_element_type=f32)
        for a in range(A)
    ]
    h2 = jnp.maximum(jnp.concatenate(parts, axis=1) + bg2_ref[...], 0.0)
    agg = lax.dot_general(h2, wgo_ref[...], (((1,), (0,)), ((), ())),
                          preferred_element_type=f32) + bgo_ref[...]
    gm = (agg[:, 0:3] + agg[:, 3:6] + agg[:, 6:9] + agg[:, 9:12]
          + agg[:, 12:15] + agg[:, 15:18] + agg[:, 18:21]
          + agg[:, 21:24]) * (1.0 / A)
    c = lax.dot_general(gm, wc1_ref[...], (((1,), (0,)), ((), ())),
                        preferred_element_type=f32) + bc1_ref[...]
    mu = jnp.mean(c, axis=-1, keepdims=True)
    var = jnp.mean((c - mu) ** 2, axis=-1, keepdims=True)
    c = (c - mu) * lax.rsqrt(var + 1e-5) * ln1g_ref[...] + ln1b_ref[...]
    c = jnp.maximum(c, 0.0)
    c = lax.dot_general(c, wc2_ref[...], (((1,), (0,)), ((), ())),
                        preferred_element_type=f32) + bc2_ref[...]
    mu = jnp.mean(c, axis=-1, keepdims=True)
    var = jnp.mean((c - mu) ** 2, axis=-1, keepdims=True)
    c = (c - mu) * lax.rsqrt(var + 1e-5) * ln2g_ref[...] + ln2b_ref[...]
    c = jnp.maximum(c, 0.0)
    out_ref[...] = lax.dot_general(c, wc3_ref[...], (((1,), (0,)), ((), ())),
                                   preferred_element_type=f32) + bc3_ref[...]


def _passC_chunk(q, h1, g2q, cnt_n, consts):
    (Wg2, bg2, Wgob, bgo, Wc1, bc1,
     ln1g, ln1b, Wc2, bc2, ln2g, ln2b, Wc3, bc3) = consts
    qb = q * (QSZ // BC)
    return pl.pallas_call(
        _passC,
        grid=(GC,),
        in_specs=[
            pl.BlockSpec((BC, 512), lambda i: (qb + i, 0)),
            pl.BlockSpec((BC, 512), lambda i: (i, 0)),
            pl.BlockSpec((BC, 1), lambda i: (qb + i, 0)),
            _full(Wg2.shape), _full(bg2.shape),
            _full(Wgob.shape), _full(bgo.shape),
            _full(Wc1.shape), _full(bc1.shape),
            _full((1, 64)), _full((1, 64)),
            _full(Wc2.shape), _full((1, 32)),
            _full((1, 32)), _full((1, 32)),
            _full(Wc3.shape), _full((1, 1)),
        ],
        out_specs=pl.BlockSpec((BC, 1), lambda i: (i, 0)),
        out_shape=jax.ShapeDtypeStruct((QSZ, 1), f32),
    )(h1, g2q, cnt_n, Wg2, bg2, Wgob, bgo, Wc1, bc1,
      ln1g, ln1b, Wc2, bc2, ln2g, ln2b, Wc3, bc3)


def kernel(x, partition_ids, W_emb1, b_emb1, W_emb2, b_emb2, W_g1, b_g1,
           W_g2, b_g2, W_go, b_go, W_c1, b_c1, ln1_g, ln1_b, W_c2, b_c2,
           ln2_g, ln2_b, W_c3, b_c3):
    x2 = x.reshape(N, A * x.shape[2]).astype(f32)
    pid32 = partition_ids.astype(jnp.int32)
    pid3 = pid32.reshape(G1, 1, B1)
    pid_pad = jnp.pad(pid32, (0, N2 - N))

    Wb1 = _blockdiag(W_emb1, A)
    bb1 = jnp.tile(b_emb1, A).reshape(1, -1)
    Wb2 = _blockdiag(W_emb2, A)
    Wb2e = jnp.concatenate([Wb2, jnp.zeros((Wb2.shape[0], 8), f32)], axis=1)
    bb2e = jnp.concatenate(
        [jnp.tile(b_emb2, A), jnp.ones((1,), f32), jnp.zeros((7,), f32)]
    ).reshape(1, 32)
    Wg1 = _blockdiag(W_g1, A)
    bg1 = jnp.tile(b_g1, A).reshape(1, -1)
    bg2 = jnp.tile(b_g2, A).reshape(1, -1)
    Wgob = _blockdiag(W_go, A)
    bgo = jnp.tile(b_go, A).reshape(1, -1)

    meas, sums1 = pl.pallas_call(
        _passA,
        grid=(G1,),
        in_specs=[
            pl.BlockSpec((B1, 128), lambda i: (i, 0)),
            pl.BlockSpec((1, 1, B1), lambda i: (i, 0, 0)),
            _full(Wb1.shape), _full(bb1.shape),
            _full(Wb2e.shape), _full(bb2e.shape),
        ],
        out_specs=[
            pl.BlockSpec((B1, 32), lambda i: (i, 0)),
            pl.BlockSpec((P, 32), lambda i: (0, 0)),
        ],
        out_shape=[
            jax.ShapeDtypeStruct((N, 32), f32),
            jax.ShapeDtypeStruct((P, 32), f32),
        ],
    )(x2, pid3, Wb1, bb1, Wb2e, bb2e)

    h1, cnt_n, sumsu = pl.pallas_call(
        _passB,
        grid=(G1,),
        in_specs=[
            pl.BlockSpec((B1, 32), lambda i: (i, 0)),
            pl.BlockSpec((1, 1, B1), lambda i: (i, 0, 0)),
            _full((P, 32)),
            _full(Wg1.shape), _full(bg1.shape),
        ],
        out_specs=[
            pl.BlockSpec((B1, 512), lambda i: (i, 0)),
            pl.BlockSpec((B1, 1), lambda i: (i, 0)),
            pl.BlockSpec((P, 512), lambda i: (0, 0)),
        ],
        out_shape=[
            jax.ShapeDtypeStruct((N2, 512), f32),
            jax.ShapeDtypeStruct((N2, 1), f32),
            jax.ShapeDtypeStruct((P, 512), f32),
        ],
    )(meas, pid3, sums1, Wg1, bg1)

    consts = (W_g2.astype(f32), bg2, Wgob, bgo,
              W_c1.astype(f32), b_c1.reshape(1, -1),
              ln1_g.reshape(1, -1), ln1_b.reshape(1, -1),
              W_c2.astype(f32), b_c2.reshape(1, -1),
              ln2_g.reshape(1, -1), ln2_b.reshape(1, -1),
              W_c3.astype(f32), b_c3.reshape(1, 1))

    outs = []
    for q in range(NQ):
        g2q = _sc_gather_chunk(sumsu, pid_pad, q)
        outs.append(_passC_chunk(q, h1, g2q, cnt_n, consts))

    out = jnp.concatenate(outs, axis=0)
    return out[:N, 0]


# speedup vs baseline: 1.3334x; 1.0814x over previous
import functools

import jax
import jax.numpy as jnp
from jax import lax
from jax.experimental import pallas as pl
from jax.experimental.pallas import tpu as pltpu
from jax.experimental.pallas import tpu_sc as plsc

N = 100000
P = 1000
A = 8

NC = 2
NS = 16
NW = NC * NS
N2 = 100352
NQ = 4
QSZ = N2 // NQ
RQ = QSZ // NW
CH = 56
NCH = RQ // CH

B1 = 2000
G1 = N // B1
BC = 3136
GC = QSZ // BC

f32 = jnp.float32


def _blockdiag(w, reps):
    return jnp.kron(jnp.eye(reps, dtype=w.dtype), w)


def _full(shape):
    return pl.BlockSpec(shape, lambda i: tuple(0 for _ in shape))


def _sc_gather_body(q):
    def body(tab_hbm, pid_hbm, out_hbm, vbuf, ibuf, sem):
        c = lax.axis_index("c")
        s = lax.axis_index("s")
        wid = s * NC + c
        row0 = pl.multiple_of(q * QSZ + wid * RQ, 8)
        loc0 = pl.multiple_of(wid * RQ, 8)

        def step(k, carry):
            off = pl.multiple_of(row0 + k * CH, 8)
            loc = pl.multiple_of(loc0 + k * CH, 8)
            pltpu.sync_copy(pid_hbm.at[pl.ds(off, CH)], ibuf)
            pltpu.async_copy(tab_hbm.at[ibuf], vbuf, sem).wait()
            pltpu.sync_copy(vbuf, out_hbm.at[pl.ds(loc, CH)])
            return carry

        lax.fori_loop(0, NCH, step, 0)

    return body


def _sc_gather_chunk(tab, pid_pad, q):
    k = pl.kernel(
        _sc_gather_body(q),
        out_type=jax.ShapeDtypeStruct((QSZ, 512), f32),
        mesh=plsc.VectorSubcoreMesh(core_axis_name="c", subcore_axis_name="s"),
        scratch_types=[
            pltpu.VMEM((CH, 512), f32),
            pltpu.VMEM((CH,), jnp.int32),
            pltpu.SemaphoreType.DMA,
        ],
        name=f"sc_gather_q{q}",
    )
    return k(tab, pid_pad)


def _exact_mask_dot(mask_bf, vals, dims):
    hi = vals.astype(jnp.bfloat16)
    r1 = vals - hi.astype(f32)
    mid = r1.astype(jnp.bfloat16)
    lo = (r1 - mid.astype(f32)).astype(jnp.bfloat16)
    out = lax.dot_general(mask_bf, hi, dims, preferred_element_type=f32)
    out += lax.dot_general(mask_bf, mid, dims, preferred_element_type=f32)
    out += lax.dot_general(mask_bf, lo, dims, preferred_element_type=f32)
    return out


def _exact2_mask_dot(mask_bf, vals, dims):
    hi = vals.astype(jnp.bfloat16)
    mid = (vals - hi.astype(f32)).astype(jnp.bfloat16)
    out = lax.dot_general(mask_bf, hi, dims, preferred_element_type=f32)
    out += lax.dot_general(mask_bf, mid, dims, preferred_element_type=f32)
    return out


def _passA(x_ref, pid_ref, wb1_ref, bb1_ref, wb2e_ref, bb2e_ref,
           meas_ref, sums1_ref):
    x = x_ref[...]
    x = jnp.where(jnp.isnan(x), 0.0, x)
    x = jnp.where(x == jnp.inf, 1.0, x)
    x = jnp.where(x == -jnp.inf, -1.0, x)
    h = jnp.maximum(
        lax.dot_general(x, wb1_ref[...], (((1,), (0,)), ((), ())),
                        preferred_element_type=f32) + bb1_ref[...], 0.0)
    meas = lax.dot_general(h, wb2e_ref[...], (((1,), (0,)), ((), ())),
                           preferred_element_type=f32) + bb2e_ref[...]
    meas_ref[...] = meas

    pid = pid_ref[0]
    iot = lax.broadcasted_iota(jnp.int32, (P, pid.shape[1]), 0)
    mask = (iot == pid).astype(jnp.bfloat16)
    part = _exact_mask_dot(mask, meas, (((1,), (0,)), ((), ())))

    @pl.when(pl.program_id(0) == 0)
    def _init():
        sums1_ref[...] = jnp.zeros_like(sums1_ref)

    sums1_ref[...] += part


def _passB(meas_ref, pid_ref, t1_ref, wg1_ref, bg1_ref,
           h1_ref, cnt_ref, sumsu_ref):
    pid = pid_ref[0]
    Bn = pid.shape[1]
    iot = lax.broadcasted_iota(jnp.int32, (P, Bn), 0)
    mask = (iot == pid).astype(jnp.bfloat16)

    g1row = _exact_mask_dot(mask, t1_ref[...],
                            (((0,), (0,)), ((), ())))
    cnt = jnp.maximum(g1row[:, 24:25], 1.0)
    pm = meas_ref[...][:, :24] + g1row[:, :24] * (1.0 / cnt)
    h1 = jnp.maximum(
        lax.dot_general(pm, wg1_ref[...], (((1,), (0,)), ((), ())),
                        preferred_element_type=f32) + bg1_ref[...], 0.0)
    h1_ref[...] = h1
    cnt_ref[...] = cnt

    part = _exact2_mask_dot(mask, h1, (((1,), (0,)), ((), ())))

    @pl.when(pl.program_id(0) == 0)
    def _init():
        sumsu_ref[...] = jnp.zeros_like(sumsu_ref)

    sumsu_ref[...] += part


def _passC(h1_ref, g2_ref, cnt_ref, wg2_ref, bg2_ref, wgo_ref, bgo_ref,
           wc1_ref, bc1_ref,
           ln1g_ref, ln1b_ref, wc2_ref, bc2_ref, ln2g_ref, ln2b_ref,
           wc3_ref, bc3_ref, out_ref):
    cnt = jnp.maximum(cnt_ref[...], 1.0)
    p2 = h1_ref[...] + g2_ref[...] * (1.0 / cnt)
    wg2 = wg2_ref[...]
    parts = [
        lax.dot_general(p2[:, 64 * a:64 * (a + 1)], wg2,
                        (((1,), (0,)), ((), ())), preferred_element_type=f32)
        for a in range(A)
    ]
    h2 = jnp.maximum(jnp.concatenate(parts, axis=1) + bg2_ref[...], 0.0)
    agg = lax.dot_general(h2, wgo_ref[...], (((1,), (0,)), ((), ())),
                          preferred_element_type=f32) + bgo_ref[...]
    gm = (agg[:, 0:3] + agg[:, 3:6] + agg[:, 6:9] + agg[:, 9:12]
          + agg[:, 12:15] + agg[:, 15:18] + agg[:, 18:21]
          + agg[:, 21:24]) * (1.0 / A)
    c = lax.dot_general(gm, wc1_ref[...], (((1,), (0,)), ((), ())),
                        preferred_element_type=f32) + bc1_ref[...]
    mu = jnp.mean(c, axis=-1, keepdims=True)
    var = jnp.mean((c - mu) ** 2, axis=-1, keepdims=True)
    c = (c - mu) * lax.rsqrt(var + 1e-5) * ln1g_ref[...] + ln1b_ref[...]
    c = jnp.maximum(c, 0.0)
    c = lax.dot_general(c, wc2_ref[...], (((1,), (0,)), ((), ())),
                        preferred_element_type=f32) + bc2_ref[...]
    mu = jnp.mean(c, axis=-1, keepdims=True)
    var = jnp.mean((c - mu) ** 2, axis=-1, keepdims=True)
    c = (c - mu) * lax.rsqrt(var + 1e-5) * ln2g_ref[...] + ln2b_ref[...]
    c = jnp.maximum(c, 0.0)
    out_ref[...] = lax.dot_general(c, wc3_ref[...], (((1,), (0,)), ((), ())),
                                   preferred_element_type=f32) + bc3_ref[...]


def _passC_chunk(q, h1, g2q, cnt_n, consts):
    (Wg2, bg2, Wgob, bgo, Wc1, bc1,
     ln1g, ln1b, Wc2, bc2, ln2g, ln2b, Wc3, bc3) = consts
    qb = q * (QSZ // BC)
    return pl.pallas_call(
        _passC,
        grid=(GC,),
        in_specs=[
            pl.BlockSpec((BC, 512), lambda i: (qb + i, 0)),
            pl.BlockSpec((BC, 512), lambda i: (i, 0)),
            pl.BlockSpec((BC, 1), lambda i: (qb + i, 0)),
            _full(Wg2.shape), _full(bg2.shape),
            _full(Wgob.shape), _full(bgo.shape),
            _full(Wc1.shape), _full(bc1.shape),
            _full((1, 64)), _full((1, 64)),
            _full(Wc2.shape), _full((1, 32)),
            _full((1, 32)), _full((1, 32)),
            _full(Wc3.shape), _full((1, 1)),
        ],
        out_specs=pl.BlockSpec((BC, 1), lambda i: (i, 0)),
        out_shape=jax.ShapeDtypeStruct((QSZ, 1), f32),
    )(h1, g2q, cnt_n, Wg2, bg2, Wgob, bgo, Wc1, bc1,
      ln1g, ln1b, Wc2, bc2, ln2g, ln2b, Wc3, bc3)


def kernel(x, partition_ids, W_emb1, b_emb1, W_emb2, b_emb2, W_g1, b_g1,
           W_g2, b_g2, W_go, b_go, W_c1, b_c1, ln1_g, ln1_b, W_c2, b_c2,
           ln2_g, ln2_b, W_c3, b_c3):
    x2 = x.reshape(N, A * x.shape[2]).astype(f32)
    pid32 = partition_ids.astype(jnp.int32)
    pid3 = pid32.reshape(G1, 1, B1)
    pid_pad = jnp.pad(pid32, (0, N2 - N))

    Wb1 = _blockdiag(W_emb1, A)
    bb1 = jnp.tile(b_emb1, A).reshape(1, -1)
    Wb2 = _blockdiag(W_emb2, A)
    Wb2e = jnp.concatenate([Wb2, jnp.zeros((Wb2.shape[0], 8), f32)], axis=1)
    bb2e = jnp.concatenate(
        [jnp.tile(b_emb2, A), jnp.ones((1,), f32), jnp.zeros((7,), f32)]
    ).reshape(1, 32)
    Wg1 = _blockdiag(W_g1, A)
    bg1 = jnp.tile(b_g1, A).reshape(1, -1)
    bg2 = jnp.tile(b_g2, A).reshape(1, -1)
    Wgob = _blockdiag(W_go, A)
    bgo = jnp.tile(b_go, A).reshape(1, -1)

    meas, sums1 = pl.pallas_call(
        _passA,
        grid=(G1,),
        in_specs=[
            pl.BlockSpec((B1, 128), lambda i: (i, 0)),
            pl.BlockSpec((1, 1, B1), lambda i: (i, 0, 0)),
            _full(Wb1.shape), _full(bb1.shape),
            _full(Wb2e.shape), _full(bb2e.shape),
        ],
        out_specs=[
            pl.BlockSpec((B1, 32), lambda i: (i, 0)),
            pl.BlockSpec((P, 32), lambda i: (0, 0)),
        ],
        out_shape=[
            jax.ShapeDtypeStruct((N, 32), f32),
            jax.ShapeDtypeStruct((P, 32), f32),
        ],
    )(x2, pid3, Wb1, bb1, Wb2e, bb2e)

    h1, cnt_n, sumsu = pl.pallas_call(
        _passB,
        grid=(G1,),
        in_specs=[
            pl.BlockSpec((B1, 32), lambda i: (i, 0)),
            pl.BlockSpec((1, 1, B1), lambda i: (i, 0, 0)),
            _full((P, 32)),
            _full(Wg1.shape), _full(bg1.shape),
        ],
        out_specs=[
            pl.BlockSpec((B1, 512), lambda i: (i, 0)),
            pl.BlockSpec((B1, 1), lambda i: (i, 0)),
            pl.BlockSpec((P, 512), lambda i: (0, 0)),
        ],
        out_shape=[
            jax.ShapeDtypeStruct((N2, 512), f32),
            jax.ShapeDtypeStruct((N2, 1), f32),
            jax.ShapeDtypeStruct((P, 512), f32),
        ],
    )(meas, pid3, sums1, Wg1, bg1)

    consts = (W_g2.astype(f32), bg2, Wgob, bgo,
              W_c1.astype(f32), b_c1.reshape(1, -1),
              ln1_g.reshape(1, -1), ln1_b.reshape(1, -1),
              W_c2.astype(f32), b_c2.reshape(1, -1),
              ln2_g.reshape(1, -1), ln2_b.reshape(1, -1),
              W_c3.astype(f32), b_c3.reshape(1, 1))

    outs = []
    for q in range(NQ):
        g2q = _sc_gather_chunk(sumsu, pid_pad, q)
        outs.append(_passC_chunk(q, h1, g2q, cnt_n, consts))

    out = jnp.concatenate(outs, axis=0)
    return out[:N, 0]
